# baseline (device time: 227561 ns/iter reference)
import jax
import jax.numpy as jnp
from jax import lax
from jax.experimental import pallas as pl
from jax.experimental.pallas import tpu as pltpu

M = 8192
N_HALF = 1024
P = 16
SR = M // P
HR = SR // 2
NHOP = P - 1
S = 2
CR = HR // S

RING = [(0, 0), (0, 1), (0, 2), (0, 3),
        (1, 3), (1, 2), (1, 1), (2, 1), (2, 2), (2, 3),
        (3, 3), (3, 2), (3, 1), (3, 0), (2, 0), (1, 0)]
STRIPE_OF_POS = [4 * y + z for (y, z) in RING]
POS_OF_STRIPE = [0] * P
for _i, _p in enumerate(STRIPE_OF_POS):
    POS_OF_STRIPE[_p] = _i
RING_Y = [y for (y, _) in RING]
RING_Z = [z for (_, z) in RING]


def _lut(table, idx):
    v = jnp.int32(table[0])
    for i in range(1, len(table)):
        v = jnp.where(idx == i, jnp.int32(table[i]), v)
    return v


def _mod16(v):
    return jnp.where(v >= P, v - P, v)


def _off(d, s):
    return d * HR + s * CR


def kernel(x):
    x = x.reshape(M, 2 * N_HALF)

    def body(x_hbm, out_v,
             cross_recv, local_buf,
             cross_send_sems, cross_recv_sems, local_sems,
             cw_send, cw_recv, ccw_send, ccw_recv, out_own_sems):
        my_x = lax.axis_index("x")
        my_y = lax.axis_index("y")
        my_z = lax.axis_index("z")
        partner = (1 - my_x, my_y, my_z)
        my_stripe = 4 * my_y + my_z
        r = _lut(POS_OF_STRIPE, my_stripe)
        rp1 = _mod16(r + 1)
        rm1 = _mod16(r + NHOP)
        nxt = (my_x, _lut(RING_Y, rp1), _lut(RING_Z, rp1))
        prv = (my_x, _lut(RING_Y, rm1), _lut(RING_Z, rm1))
        cw_rslot = [_lut(STRIPE_OF_POS, _mod16(r + NHOP - h)) for h in range(NHOP)]
        ccw_rslot = [_lut(STRIPE_OF_POS, _mod16(_mod16(r + 1 + h))) for h in range(NHOP)]
        row0 = my_stripe * SR

        barrier = pltpu.get_barrier_semaphore()
        for dev in (partner, nxt, prv):
            pl.semaphore_signal(
                barrier, inc=1, device_id=dev,
                device_id_type=pl.DeviceIdType.MESH,
            )
        pl.semaphore_wait(barrier, 3)

        tgt = (nxt, prv)

        crosses = []
        locals_ = []
        for d in range(2):
            for s in range(S):
                i = d * S + s
                off = _off(d, s)
                c = pltpu.make_async_remote_copy(
                    src_ref=x_hbm.at[pl.ds(row0 + off, CR),
                                     pl.ds((1 - my_x) * N_HALF, N_HALF)],
                    dst_ref=cross_recv.at[pl.ds(off, CR), :],
                    send_sem=cross_send_sems.at[i],
                    recv_sem=cross_recv_sems.at[i],
                    device_id=partner,
                    device_id_type=pl.DeviceIdType.MESH,
                )
                c.start()
                lc = pltpu.make_async_copy(
                    x_hbm.at[pl.ds(row0 + off, CR),
                             pl.ds(my_x * N_HALF, N_HALF)],
                    local_buf.at[pl.ds(off, CR), :],
                    local_sems.at[i],
                )
                lc.start()
                crosses.append(c)
                locals_.append(lc)

        sends = []
        sem_send = (cw_send, ccw_send)
        sem_recv = (cw_recv, ccw_recv)
        for d in range(2):
            for s in range(S):
                i = d * S + s
                off = _off(d, s)
                locals_[i].wait()
                crosses[i].wait_recv()
                sub = pl.ds(off, CR)
                local_buf[sub, :] = local_buf[sub, :] + cross_recv[sub, :]
                h0 = pltpu.make_async_remote_copy(
                    src_ref=local_buf.at[sub, :],
                    dst_ref=out_v.at[pl.ds(row0 + off, CR), :],
                    send_sem=sem_send[d].at[0, s],
                    recv_sem=sem_recv[d].at[0, s],
                    device_id=tgt[d],
                    device_id_type=pl.DeviceIdType.MESH,
                )
                h0.start()
                sends.append(h0)
                oo = pltpu.make_async_copy(
                    local_buf.at[sub, :],
                    out_v.at[pl.ds(row0 + off, CR), :],
                    out_own_sems.at[i],
                )
                oo.start()

        rslot = (cw_rslot, ccw_rslot)
        for h in range(NHOP):
            for s in range(S):
                for d in range(2):
                    rows = pl.ds(rslot[d][h] * SR + _off(d, s), CR)
                    rc = pltpu.make_async_remote_copy(
                        src_ref=out_v.at[rows, :],
                        dst_ref=out_v.at[rows, :],
                        send_sem=sem_send[d].at[h, s],
                        recv_sem=sem_recv[d].at[h, s],
                        device_id=tgt[d],
                        device_id_type=pl.DeviceIdType.MESH,
                    )
                    rc.wait_recv()
                    if h + 1 < NHOP:
                        fw = pltpu.make_async_remote_copy(
                            src_ref=out_v.at[rows, :],
                            dst_ref=out_v.at[rows, :],
                            send_sem=sem_send[d].at[h + 1, s],
                            recv_sem=sem_recv[d].at[h + 1, s],
                            device_id=tgt[d],
                            device_id_type=pl.DeviceIdType.MESH,
                        )
                        fw.start()
                        sends.append(fw)

        for i in range(2 * S):
            pltpu.make_async_copy(
                local_buf.at[pl.ds(0, CR), :],
                out_v.at[pl.ds(0, CR), :],
                out_own_sems.at[i],
            ).wait()
        for c in crosses:
            c.wait_send()
        for snd in sends:
            snd.wait_send()

    return pl.pallas_call(
        body,
        out_shape=jax.ShapeDtypeStruct((M, N_HALF), jnp.float32),
        in_specs=[pl.BlockSpec(memory_space=pl.ANY)],
        out_specs=pl.BlockSpec(memory_space=pltpu.MemorySpace.VMEM),
        scratch_shapes=[
            pltpu.VMEM((SR, N_HALF), jnp.float32),
            pltpu.VMEM((SR, N_HALF), jnp.float32),
            pltpu.SemaphoreType.DMA((2 * S,)),
            pltpu.SemaphoreType.DMA((2 * S,)),
            pltpu.SemaphoreType.DMA((2 * S,)),
            pltpu.SemaphoreType.DMA((NHOP, S)),
            pltpu.SemaphoreType.DMA((NHOP, S)),
            pltpu.SemaphoreType.DMA((NHOP, S)),
            pltpu.SemaphoreType.DMA((NHOP, S)),
            pltpu.SemaphoreType.DMA((2 * S,)),
        ],
        compiler_params=pltpu.CompilerParams(
            collective_id=0,
            vmem_limit_bytes=56 * 1024 * 1024,
        ),
    )(x)


# device time: 219101 ns/iter; 1.0386x vs baseline; 1.0386x over previous
import jax
import jax.numpy as jnp
from jax import lax
from jax.experimental import pallas as pl
from jax.experimental.pallas import tpu as pltpu

M = 8192
N_HALF = 1024
P = 16
SR = M // P
HR = SR // 2
NHOP = P - 1
S = 2
CR = HR // S

RING = [(0, 0), (0, 1), (0, 2), (0, 3),
        (1, 3), (1, 2), (1, 1), (2, 1), (2, 2), (2, 3),
        (3, 3), (3, 2), (3, 1), (3, 0), (2, 0), (1, 0)]
STRIPE_OF_POS = [4 * y + z for (y, z) in RING]
POS_OF_STRIPE = [0] * P
for _i, _p in enumerate(STRIPE_OF_POS):
    POS_OF_STRIPE[_p] = _i
RING_Y = [y for (y, _) in RING]
RING_Z = [z for (_, z) in RING]


def _lut(table, idx):
    v = jnp.int32(table[0])
    for i in range(1, len(table)):
        v = jnp.where(idx == i, jnp.int32(table[i]), v)
    return v


def _mod16(v):
    return jnp.where(v >= P, v - P, v)


def _off(d, s):
    return d * HR + s * CR


def kernel(x):
    x = x.reshape(M, 2 * N_HALF)

    def body(x_hbm, out_hbm,
             plane, cross_recv, local_buf,
             cross_send_sems, cross_recv_sems, local_sems,
             cw_send, cw_recv, ccw_send, ccw_recv,
             out_cw_sems, out_ccw_sems, out_own_sems):
        my_x = lax.axis_index("x")
        my_y = lax.axis_index("y")
        my_z = lax.axis_index("z")
        partner = (1 - my_x, my_y, my_z)
        my_stripe = 4 * my_y + my_z
        r = _lut(POS_OF_STRIPE, my_stripe)
        rp1 = _mod16(r + 1)
        rm1 = _mod16(r + NHOP)
        nxt = (my_x, _lut(RING_Y, rp1), _lut(RING_Z, rp1))
        prv = (my_x, _lut(RING_Y, rm1), _lut(RING_Z, rm1))
        cw_rslot = [_lut(STRIPE_OF_POS, _mod16(r + NHOP - h)) for h in range(NHOP)]
        ccw_rslot = [_lut(STRIPE_OF_POS, _mod16(_mod16(r + 1 + h))) for h in range(NHOP)]
        row0 = my_stripe * SR

        barrier = pltpu.get_barrier_semaphore()
        for dev in (partner, nxt, prv):
            pl.semaphore_signal(
                barrier, inc=1, device_id=dev,
                device_id_type=pl.DeviceIdType.MESH,
            )
        pl.semaphore_wait(barrier, 3)

        tgt = (nxt, prv)

        crosses = []
        locals_ = []
        for d in range(2):
            for s in range(S):
                i = d * S + s
                off = _off(d, s)
                c = pltpu.make_async_remote_copy(
                    src_ref=x_hbm.at[pl.ds(row0 + off, CR),
                                     pl.ds((1 - my_x) * N_HALF, N_HALF)],
                    dst_ref=cross_recv.at[pl.ds(off, CR), :],
                    send_sem=cross_send_sems.at[i],
                    recv_sem=cross_recv_sems.at[i],
                    device_id=partner,
                    device_id_type=pl.DeviceIdType.MESH,
                )
                c.start()
                lc = pltpu.make_async_copy(
                    x_hbm.at[pl.ds(row0 + off, CR),
                             pl.ds(my_x * N_HALF, N_HALF)],
                    local_buf.at[pl.ds(off, CR), :],
                    local_sems.at[i],
                )
                lc.start()
                crosses.append(c)
                locals_.append(lc)

        sends = []
        sem_send = (cw_send, ccw_send)
        sem_recv = (cw_recv, ccw_recv)
        for d in range(2):
            for s in range(S):
                i = d * S + s
                off = _off(d, s)
                locals_[i].wait()
                crosses[i].wait_recv()
                sub = pl.ds(off, CR)
                local_buf[sub, :] = local_buf[sub, :] + cross_recv[sub, :]
                h0 = pltpu.make_async_remote_copy(
                    src_ref=local_buf.at[sub, :],
                    dst_ref=plane.at[my_stripe, sub, :],
                    send_sem=sem_send[d].at[0, s],
                    recv_sem=sem_recv[d].at[0, s],
                    device_id=tgt[d],
                    device_id_type=pl.DeviceIdType.MESH,
                )
                h0.start()
                sends.append(h0)
                oo = pltpu.make_async_copy(
                    local_buf.at[sub, :],
                    out_hbm.at[pl.ds(row0 + off, CR), :],
                    out_own_sems.at[i],
                )
                oo.start()

        rslot = (cw_rslot, ccw_rslot)
        out_sems = (out_cw_sems, out_ccw_sems)
        for h in range(NHOP):
            for s in range(S):
                for d in range(2):
                    slot = rslot[d][h]
                    sub = pl.ds(_off(d, s), CR)
                    rc = pltpu.make_async_remote_copy(
                        src_ref=plane.at[slot, sub, :],
                        dst_ref=plane.at[slot, sub, :],
                        send_sem=sem_send[d].at[h, s],
                        recv_sem=sem_recv[d].at[h, s],
                        device_id=tgt[d],
                        device_id_type=pl.DeviceIdType.MESH,
                    )
                    rc.wait_recv()
                    if h + 1 < NHOP:
                        fw = pltpu.make_async_remote_copy(
                            src_ref=plane.at[slot, sub, :],
                            dst_ref=plane.at[slot, sub, :],
                            send_sem=sem_send[d].at[h + 1, s],
                            recv_sem=sem_recv[d].at[h + 1, s],
                            device_id=tgt[d],
                            device_id_type=pl.DeviceIdType.MESH,
                        )
                        fw.start()
                        sends.append(fw)
                    od = pltpu.make_async_copy(
                        plane.at[slot, sub, :],
                        out_hbm.at[pl.ds(slot * SR + _off(d, s), CR), :],
                        out_sems[d].at[h, s],
                    )
                    od.start()

        for i in range(2 * S):
            pltpu.make_async_copy(
                local_buf.at[pl.ds(0, CR), :],
                out_hbm.at[pl.ds(0, CR), :],
                out_own_sems.at[i],
            ).wait()
        for h in range(NHOP):
            for s in range(S):
                for d in range(2):
                    pltpu.make_async_copy(
                        plane.at[0, pl.ds(0, CR), :],
                        out_hbm.at[pl.ds(0, CR), :],
                        out_sems[d].at[h, s],
                    ).wait()
        for c in crosses:
            c.wait_send()
        for snd in sends:
            snd.wait_send()

    return pl.pallas_call(
        body,
        out_shape=jax.ShapeDtypeStruct((M, N_HALF), jnp.float32),
        in_specs=[pl.BlockSpec(memory_space=pl.ANY)],
        out_specs=pl.BlockSpec(memory_space=pl.ANY),
        scratch_shapes=[
            pltpu.VMEM((P, SR, N_HALF), jnp.float32),
            pltpu.VMEM((SR, N_HALF), jnp.float32),
            pltpu.VMEM((SR, N_HALF), jnp.float32),
            pltpu.SemaphoreType.DMA((2 * S,)),
            pltpu.SemaphoreType.DMA((2 * S,)),
            pltpu.SemaphoreType.DMA((2 * S,)),
            pltpu.SemaphoreType.DMA((NHOP, S)),
            pltpu.SemaphoreType.DMA((NHOP, S)),
            pltpu.SemaphoreType.DMA((NHOP, S)),
            pltpu.SemaphoreType.DMA((NHOP, S)),
            pltpu.SemaphoreType.DMA((NHOP, S)),
            pltpu.SemaphoreType.DMA((NHOP, S)),
            pltpu.SemaphoreType.DMA((2 * S,)),
        ],
        compiler_params=pltpu.CompilerParams(
            collective_id=0,
            vmem_limit_bytes=56 * 1024 * 1024,
        ),
    )(x)
